# baseline (speedup 1.0000x reference)
import functools

import jax
import jax.numpy as jnp
from jax import lax
from jax.experimental import pallas as pl
from jax.experimental.pallas import tpu as pltpu
from jax.experimental.pallas import tpu_sc as plsc

_DEFAULT_VALUE = -1
_CHUNK = 128


def kernel(labels, embedding_table):
    (B,) = labels.shape
    V, D = embedding_table.shape
    info = plsc.get_sparse_core_info()
    num_workers = info.num_cores * info.num_subcores
    b_per_w = B // num_workers
    n_chunks = b_per_w // _CHUNK
    lanes = info.num_lanes
    mesh = plsc.VectorSubcoreMesh(core_axis_name="c", subcore_axis_name="s")

    @functools.partial(
        pl.kernel,
        mesh=mesh,
        out_type=jax.ShapeDtypeStruct((B, D), jnp.float32),
        compiler_params=pltpu.CompilerParams(use_tc_tiling_on_sc=False),
        scratch_types=[
            pltpu.VMEM((n_chunks, _CHUNK), jnp.int32),
            pltpu.VMEM((n_chunks, _CHUNK, D), jnp.float32),
            pltpu.SemaphoreType.DMA,
        ],
    )
    def _embed(labels_hbm, table_hbm, out_hbm, idx_v, rows_v, sem):
        wid = lax.axis_index("s") * info.num_cores + lax.axis_index("c")
        base = wid * b_per_w
        for c in range(n_chunks):
            pltpu.sync_copy(labels_hbm.at[pl.ds(base + c * _CHUNK, _CHUNK)],
                            idx_v.at[c])
        for c in range(n_chunks):
            for i in range(_CHUNK // lanes):
                s = idx_v[c, pl.ds(i * lanes, lanes)]
                idx_v[c, pl.ds(i * lanes, lanes)] = jnp.where(
                    s == _DEFAULT_VALUE, V - 1, s)
        copies = [
            pltpu.async_copy(table_hbm.at[idx_v.at[c]], rows_v.at[c], sem)
            for c in range(n_chunks)
        ]
        for cp in copies:
            cp.wait()
        for c in range(n_chunks):
            pltpu.sync_copy(rows_v.at[c],
                            out_hbm.at[pl.ds(base + c * _CHUNK, _CHUNK)])

    return _embed(labels.astype(jnp.int32), embedding_table)


# speedup vs baseline: 2.2560x; 2.2560x over previous
import functools

import jax
import jax.numpy as jnp
from jax import lax
from jax.experimental import pallas as pl
from jax.experimental.pallas import tpu as pltpu
from jax.experimental.pallas import tpu_sc as plsc


def kernel(labels, embedding_table):
    (B,) = labels.shape
    V, D = embedding_table.shape
    n_groups = (V - 1) // 8
    table3 = embedding_table[: n_groups * 8].reshape(n_groups, 8, D)
    null_row = embedding_table[V - 1]

    s = labels.astype(jnp.int32)
    sel = jnp.where(s < 0, s + V, s)
    sel = jnp.clip(sel, 0, V - 1)
    k_arr = jnp.minimum(sel >> 3, n_groups - 1)
    renc_arr = (sel & 7) + jnp.where(sel == V - 1, 16, 0)

    info = plsc.get_sparse_core_info()
    num_workers = info.num_cores * info.num_subcores
    b_per_w = B // num_workers
    L = info.num_lanes
    n_grp = b_per_w // L
    mesh = plsc.VectorSubcoreMesh(core_axis_name="c", subcore_axis_name="s")

    @functools.partial(
        pl.kernel,
        mesh=mesh,
        out_type=jax.ShapeDtypeStruct((B, D), jnp.float32),
        compiler_params=pltpu.CompilerParams(use_tc_tiling_on_sc=True),
        scratch_types=[
            pltpu.VMEM((b_per_w,), jnp.int32),
            pltpu.VMEM((b_per_w,), jnp.int32),
            pltpu.VMEM((2 * L, 8, D), jnp.float32),
            pltpu.VMEM((b_per_w, D), jnp.float32),
            pltpu.VMEM((D,), jnp.float32),
            pltpu.SemaphoreType.DMA,
            pltpu.SemaphoreType.DMA,
        ],
    )
    def _embed(k_hbm, renc_hbm, table_hbm, null_hbm, out_hbm,
               k_v, renc_v, rows_v, out_v, null_v, sem_a, sem_b):
        wid = lax.axis_index("s") * info.num_cores + lax.axis_index("c")
        base = wid * b_per_w
        pltpu.sync_copy(null_hbm, null_v)
        pltpu.sync_copy(k_hbm.at[pl.ds(base, b_per_w)], k_v)
        pltpu.sync_copy(renc_hbm.at[pl.ds(base, b_per_w)], renc_v)

        def fire(g, sem, slot0):
            kv = k_v[pl.ds(g * L, L)]
            for l in range(L):
                pltpu.async_copy(table_hbm.at[pl.ds(kv[l], 1)],
                                 rows_v.at[pl.ds(slot0 + l, 1)], sem)

        def drain(sem):
            for _ in range(L):
                pltpu.make_async_copy(table_hbm.at[pl.ds(0, 1)],
                                      rows_v.at[pl.ds(0, 1)], sem).wait()

        def extract(g, slot0):
            rv = renc_v[pl.ds(g * L, L)]
            for l in range(L):
                re = rv[l]
                r = re & 7
                mv = jnp.broadcast_to(
                    jnp.where(re >= 16, 1.0, 0.0), (L,))
                j = g * L + l
                for q in range(D // L):
                    d = rows_v[slot0 + l, r, pl.ds(q * L, L)]
                    n = null_v[pl.ds(q * L, L)]
                    out_v[j, pl.ds(q * L, L)] = d + (n - d) * mv

        fire(0, sem_a, 0)
        fire(1, sem_b, L)
        drain(sem_a)
        extract(0, 0)

        def body(t, _):
            g0 = 2 * t
            fire(g0, sem_a, 0)
            drain(sem_b)
            extract(g0 - 1, L)
            fire(g0 + 1, sem_b, L)
            drain(sem_a)
            extract(g0, 0)
            return _

        lax.fori_loop(1, n_grp // 2, body, None)
        drain(sem_b)
        extract(n_grp - 1, L)
        pltpu.sync_copy(out_v, out_hbm.at[pl.ds(base, b_per_w)])

    return _embed(k_arr, renc_arr, table3, null_row)
